# baseline (speedup 1.0000x reference)
import functools

import jax
import jax.numpy as jnp
from jax import lax
from jax.experimental import pallas as pl
from jax.experimental.pallas import tpu as pltpu
from jax.experimental.pallas import tpu_sc as plsc

VOCAB = 100000
EMB = 16
BATCH = 1024
CTX = 20

NC = 2
NS = 16
NW = NC * NS
BPW = BATCH // NW
IPW = BPW * CTX
CHUNK = 128
NCH = IPW // CHUNK

BV = 2048
NV = (VOCAB + BV - 1) // BV


def _sc_gather_mean(idx3, table):
    mesh = plsc.VectorSubcoreMesh(core_axis_name="c", subcore_axis_name="s")

    @functools.partial(
        pl.kernel,
        mesh=mesh,
        out_type=jax.ShapeDtypeStruct((BATCH, EMB), jnp.float32),
        scratch_types=[
            pltpu.VMEM((NCH, CHUNK), jnp.int32),
            pltpu.VMEM((IPW, EMB), jnp.float32),
            pltpu.VMEM((BPW, EMB), jnp.float32),
            pltpu.SemaphoreType.DMA,
        ],
        compiler_params=pltpu.CompilerParams(use_tc_tiling_on_sc=False),
    )
    def k(idx_hbm, table_hbm, out_hbm, idx_v, rows_v, acc_v, sem):
        wid = lax.axis_index("s") * NC + lax.axis_index("c")
        pltpu.sync_copy(idx_hbm.at[wid], idx_v)
        copies = [
            pltpu.async_copy(
                table_hbm.at[idx_v.at[c]],
                rows_v.at[pl.ds(c * CHUNK, CHUNK)],
                sem,
            )
            for c in range(NCH)
        ]
        for cp in copies:
            cp.wait()
        for b in range(BPW):
            acc = rows_v[b * CTX, :]
            for j in range(1, CTX):
                acc = acc + rows_v[b * CTX + j, :]
            acc_v[b, :] = acc * (1.0 / CTX)
        pltpu.sync_copy(acc_v, out_hbm.at[pl.ds(wid * BPW, BPW)])

    return k(idx3, table)


def _p1_body(h_ref, w_ref, b_ref, lse_ref, m_ref, s_ref):
    j = pl.program_id(0)

    @pl.when(j == 0)
    def _():
        m_ref[...] = jnp.full((BATCH, 1), -jnp.inf, jnp.float32)
        s_ref[...] = jnp.zeros((BATCH, 1), jnp.float32)

    logits = lax.dot_general(
        h_ref[...], w_ref[...], (((1,), (1,)), ((), ())),
        preferred_element_type=jnp.float32,
    ) + b_ref[...]
    col = j * BV + lax.broadcasted_iota(jnp.int32, (1, BV), 1)
    logits = jnp.where(col < VOCAB, logits, -jnp.inf)

    bm = jnp.max(logits, axis=1, keepdims=True)
    m_old = m_ref[...]
    m_new = jnp.maximum(m_old, bm)
    s_ref[...] = s_ref[...] * jnp.exp(m_old - m_new) + jnp.sum(
        jnp.exp(logits - m_new), axis=1, keepdims=True)
    m_ref[...] = m_new

    @pl.when(j == NV - 1)
    def _():
        lse_ref[...] = m_ref[...] + jnp.log(s_ref[...])


def _p2_body(h_ref, w_ref, b_ref, lse_ref, o_ref):
    logits = lax.dot_general(
        h_ref[...], w_ref[...], (((1,), (1,)), ((), ())),
        preferred_element_type=jnp.float32,
    ) + b_ref[...]
    o_ref[...] = logits - lse_ref[...]


def _logsoftmax_linear(hidden, W, b2):
    common_in = [
        pl.BlockSpec((BATCH, EMB), lambda j: (0, 0)),
        pl.BlockSpec((BV, EMB), lambda j: (j, 0)),
        pl.BlockSpec((1, BV), lambda j: (0, j)),
    ]
    lse = pl.pallas_call(
        _p1_body,
        grid=(NV,),
        in_specs=common_in,
        out_specs=pl.BlockSpec((BATCH, 1), lambda j: (0, 0)),
        out_shape=jax.ShapeDtypeStruct((BATCH, 1), jnp.float32),
        scratch_shapes=[
            pltpu.VMEM((BATCH, 1), jnp.float32),
            pltpu.VMEM((BATCH, 1), jnp.float32),
        ],
    )(hidden, W, b2)
    out = pl.pallas_call(
        _p2_body,
        grid=(NV,),
        in_specs=common_in + [pl.BlockSpec((BATCH, 1), lambda j: (0, 0))],
        out_specs=pl.BlockSpec((BATCH, BV), lambda j: (0, j)),
        out_shape=jax.ShapeDtypeStruct((BATCH, VOCAB), jnp.float32),
    )(hidden, W, b2, lse)
    return out


def kernel(center_word_idx, emb_table, W, b):
    idx3 = center_word_idx.astype(jnp.int32).reshape(NW, NCH, CHUNK)
    hidden = _sc_gather_mean(idx3, emb_table)
    return _logsoftmax_linear(hidden, W, b.reshape(1, VOCAB))
